# initial kernel scaffold (unmeasured)
import functools

import jax
import jax.numpy as jnp
from jax import lax
from jax.experimental import pallas as pl
from jax.experimental.pallas import tpu as pltpu

M = 4096
NOUT = 4096
XS, YS, ZS = 2, 4, 4
NBLK = YS * ZS
BLK = M // NBLK
SLAB = M // YS


def kernel(dy, W):
    my_y = lax.axis_index("y")
    my_z = lax.axis_index("z")
    r = my_y * ZS + my_z
    dy_rows = lax.dynamic_slice(dy, (r * BLK, 0), (BLK, dy.shape[1]))
    partial = jnp.einsum(
        "mk,nk->mn", dy_rows, W, preferred_element_type=jnp.float32
    )

    def body(p_ref, out_ref, xrecv, send_sems, recv_sems):
        x = lax.axis_index("x")
        y = lax.axis_index("y")
        z = lax.axis_index("z")
        blk = y * ZS + z

        neighbors = [
            (1 - x, y, z),
            (x, y, (z + 1) % ZS),
            (x, y, (z - 1) % ZS),
            (x, (y + 1) % YS, z),
            (x, (y - 1) % YS, z),
        ]

        barrier = pltpu.get_barrier_semaphore()
        for nbr in neighbors:
            pl.semaphore_signal(
                barrier, inc=1, device_id=nbr,
                device_id_type=pl.DeviceIdType.MESH,
            )
        pl.semaphore_wait(barrier, len(neighbors))

        xchg = pltpu.make_async_remote_copy(
            src_ref=p_ref,
            dst_ref=xrecv,
            send_sem=send_sems.at[0],
            recv_sem=recv_sems.at[0],
            device_id=(1 - x, y, z),
            device_id_type=pl.DeviceIdType.MESH,
        )
        xchg.start()
        xchg.wait()
        out_ref[pl.ds(blk * BLK, BLK), :] = p_ref[...] + xrecv[...]

        for h in range(ZS - 1):
            z_origin = (z - h) % ZS
            row0 = y * SLAB + z_origin * BLK
            rd = pltpu.make_async_remote_copy(
                src_ref=out_ref.at[pl.ds(row0, BLK), :],
                dst_ref=out_ref.at[pl.ds(row0, BLK), :],
                send_sem=send_sems.at[1 + h],
                recv_sem=recv_sems.at[1 + h],
                device_id=(x, y, (z + 1) % ZS),
                device_id_type=pl.DeviceIdType.MESH,
            )
            rd.start()
            rd.wait()

        for h in range(YS - 1):
            y_origin = (y - h) % YS
            row0 = y_origin * SLAB
            rd = pltpu.make_async_remote_copy(
                src_ref=out_ref.at[pl.ds(row0, SLAB), :],
                dst_ref=out_ref.at[pl.ds(row0, SLAB), :],
                send_sem=send_sems.at[4 + h],
                recv_sem=recv_sems.at[4 + h],
                device_id=(x, (y + 1) % YS, z),
                device_id_type=pl.DeviceIdType.MESH,
            )
            rd.start()
            rd.wait()

        @functools.partial(pl.run_scoped, sem=pltpu.SemaphoreType.REGULAR)
        def _(sem):
            for nbr in neighbors:
                pl.semaphore_signal(
                    sem, inc=1, device_id=nbr,
                    device_id_type=pl.DeviceIdType.MESH,
                )
            pl.semaphore_wait(sem, len(neighbors))

    return pl.pallas_call(
        body,
        out_shape=jax.ShapeDtypeStruct((M, NOUT), jnp.float32),
        in_specs=[pl.BlockSpec(memory_space=pltpu.VMEM)],
        out_specs=pl.BlockSpec(memory_space=pltpu.VMEM),
        scratch_shapes=[
            pltpu.VMEM((BLK, NOUT), jnp.float32),
            pltpu.SemaphoreType.DMA((7,)),
            pltpu.SemaphoreType.DMA((7,)),
        ],
        compiler_params=pltpu.CompilerParams(collective_id=0),
    )(partial)


# baseline (device time: 834983 ns/iter reference)
import functools

import jax
import jax.numpy as jnp
from jax import lax
from jax.experimental import pallas as pl
from jax.experimental.pallas import tpu as pltpu

M = 4096
NOUT = 4096
XS, YS, ZS = 2, 4, 4
NBLK = YS * ZS
BLK = M // NBLK
SLAB = M // YS


def kernel(dy, W):
    my_y = lax.axis_index("y")
    my_z = lax.axis_index("z")
    r = my_y * ZS + my_z
    dy_rows = lax.dynamic_slice(dy, (r * BLK, 0), (BLK, dy.shape[1]))
    partial = jnp.einsum(
        "mk,nk->mn", dy_rows, W, preferred_element_type=jnp.float32
    )

    def body(p_ref, out_ref, xrecv, sumbuf, send_sems, recv_sems, copy_sem):
        x = lax.axis_index("x")
        y = lax.axis_index("y")
        z = lax.axis_index("z")
        blk = y * ZS + z

        neighbors = [
            (1 - x, y, z),
            (x, y, (z + 1) % ZS),
            (x, y, (z - 1) % ZS),
            (x, (y + 1) % YS, z),
            (x, (y - 1) % YS, z),
        ]

        barrier = pltpu.get_barrier_semaphore()
        for nbr in neighbors:
            pl.semaphore_signal(
                barrier, inc=1, device_id=nbr,
                device_id_type=pl.DeviceIdType.MESH,
            )
        pl.semaphore_wait(barrier, len(neighbors))

        xchg = pltpu.make_async_remote_copy(
            src_ref=p_ref,
            dst_ref=xrecv,
            send_sem=send_sems.at[0],
            recv_sem=recv_sems.at[0],
            device_id=(1 - x, y, z),
            device_id_type=pl.DeviceIdType.MESH,
        )
        xchg.start()
        xchg.wait()
        sumbuf[...] = p_ref[...] + xrecv[...]
        cp = pltpu.make_async_copy(
            sumbuf, out_ref.at[pl.ds(blk * BLK, BLK), :], copy_sem
        )
        cp.start()
        cp.wait()

        for h in range(ZS - 1):
            z_origin = (z - h) % ZS
            row0 = y * SLAB + z_origin * BLK
            rd = pltpu.make_async_remote_copy(
                src_ref=out_ref.at[pl.ds(row0, BLK), :],
                dst_ref=out_ref.at[pl.ds(row0, BLK), :],
                send_sem=send_sems.at[1 + h],
                recv_sem=recv_sems.at[1 + h],
                device_id=(x, y, (z + 1) % ZS),
                device_id_type=pl.DeviceIdType.MESH,
            )
            rd.start()
            rd.wait()

        for h in range(YS - 1):
            y_origin = (y - h) % YS
            row0 = y_origin * SLAB
            rd = pltpu.make_async_remote_copy(
                src_ref=out_ref.at[pl.ds(row0, SLAB), :],
                dst_ref=out_ref.at[pl.ds(row0, SLAB), :],
                send_sem=send_sems.at[4 + h],
                recv_sem=recv_sems.at[4 + h],
                device_id=(x, (y + 1) % YS, z),
                device_id_type=pl.DeviceIdType.MESH,
            )
            rd.start()
            rd.wait()

        @functools.partial(pl.run_scoped, sem=pltpu.SemaphoreType.REGULAR)
        def _(sem):
            for nbr in neighbors:
                pl.semaphore_signal(
                    sem, inc=1, device_id=nbr,
                    device_id_type=pl.DeviceIdType.MESH,
                )
            pl.semaphore_wait(sem, len(neighbors))

    return pl.pallas_call(
        body,
        out_shape=jax.ShapeDtypeStruct((M, NOUT), jnp.float32),
        in_specs=[pl.BlockSpec(memory_space=pltpu.VMEM)],
        out_specs=pl.BlockSpec(memory_space=pl.ANY),
        scratch_shapes=[
            pltpu.VMEM((BLK, NOUT), jnp.float32),
            pltpu.VMEM((BLK, NOUT), jnp.float32),
            pltpu.SemaphoreType.DMA((7,)),
            pltpu.SemaphoreType.DMA((7,)),
            pltpu.SemaphoreType.DMA,
        ],
        compiler_params=pltpu.CompilerParams(collective_id=0),
    )(partial)


# device time: 544671 ns/iter; 1.5330x vs baseline; 1.5330x over previous
import functools

import jax
import jax.numpy as jnp
from jax import lax
from jax.experimental import pallas as pl
from jax.experimental.pallas import tpu as pltpu

M = 4096
NOUT = 4096
XS, YS, ZS = 2, 4, 4
NBLK = YS * ZS
BLK = M // NBLK
SLAB = M // YS
HALF = NOUT // 2
SUB = SLAB // 2

S_X = 0
S_Z = 1
S_FZ = 4
S_Y = 7
S_FY = 13
NSEM = 19


def kernel(dy, W):
    my_y = lax.axis_index("y")
    my_z = lax.axis_index("z")
    r = my_y * ZS + my_z
    dy_rows = lax.dynamic_slice(dy, (r * BLK, 0), (BLK, dy.shape[1]))
    partial = jnp.einsum(
        "mk,nk->mn", dy_rows, W, preferred_element_type=jnp.float32
    )

    def body(p_ref, out_ref, xrecv, sumbuf, send_sems, recv_sems, copy_sem):
        x = lax.axis_index("x")
        y = lax.axis_index("y")
        z = lax.axis_index("z")
        blk = y * ZS + z
        c0 = x * HALF
        partner = (1 - x, y, z)

        neighbors = [
            partner,
            (x, y, (z + 1) % ZS),
            (x, y, (z - 1) % ZS),
            (x, (y + 1) % YS, z),
            (x, (y - 1) % YS, z),
        ]

        barrier = pltpu.get_barrier_semaphore()
        for nbr in neighbors:
            pl.semaphore_signal(
                barrier, inc=1, device_id=nbr,
                device_id_type=pl.DeviceIdType.MESH,
            )
        pl.semaphore_wait(barrier, len(neighbors))

        def rdma(src, dst, slot, dev):
            return pltpu.make_async_remote_copy(
                src_ref=src, dst_ref=dst,
                send_sem=send_sems.at[slot], recv_sem=recv_sems.at[slot],
                device_id=dev, device_id_type=pl.DeviceIdType.MESH,
            )

        xchg = rdma(p_ref, xrecv, S_X, partner)
        xchg.start()
        xchg.wait()
        sumbuf[...] = p_ref[...] + xrecv[...]
        cp = pltpu.make_async_copy(
            sumbuf, out_ref.at[pl.ds(blk * BLK, BLK), :], copy_sem
        )
        cp.start()
        cp.wait()

        fwds = []

        for h in range(ZS - 1):
            row0 = y * SLAB + ((z - h) % ZS) * BLK
            rd = rdma(
                out_ref.at[pl.ds(row0, BLK), pl.ds(c0, HALF)],
                out_ref.at[pl.ds(row0, BLK), pl.ds(c0, HALF)],
                S_Z + h, (x, y, (z + 1) % ZS),
            )
            rd.start()
            rd.wait()
            rrow0 = y * SLAB + ((z - h - 1) % ZS) * BLK
            fw = rdma(
                out_ref.at[pl.ds(rrow0, BLK), pl.ds(c0, HALF)],
                out_ref.at[pl.ds(rrow0, BLK), pl.ds(c0, HALF)],
                S_FZ + h, partner,
            )
            fw.start()
            fwds.append(fw)

        for h in range(YS - 1):
            y_o = (y - h) % YS
            rds = []
            for s in range(2):
                row0 = y_o * SLAB + s * SUB
                rd = rdma(
                    out_ref.at[pl.ds(row0, SUB), pl.ds(c0, HALF)],
                    out_ref.at[pl.ds(row0, SUB), pl.ds(c0, HALF)],
                    S_Y + 2 * h + s, (x, (y + 1) % YS, z),
                )
                rd.start()
                rds.append(rd)
            for s in range(2):
                rds[s].wait()
                rrow0 = ((y - h - 1) % YS) * SLAB + s * SUB
                fw = rdma(
                    out_ref.at[pl.ds(rrow0, SUB), pl.ds(c0, HALF)],
                    out_ref.at[pl.ds(rrow0, SUB), pl.ds(c0, HALF)],
                    S_FY + 2 * h + s, partner,
                )
                fw.start()
                fwds.append(fw)

        for fw in fwds:
            fw.wait()

        @functools.partial(pl.run_scoped, sem=pltpu.SemaphoreType.REGULAR)
        def _(sem):
            for nbr in neighbors:
                pl.semaphore_signal(
                    sem, inc=1, device_id=nbr,
                    device_id_type=pl.DeviceIdType.MESH,
                )
            pl.semaphore_wait(sem, len(neighbors))

    return pl.pallas_call(
        body,
        out_shape=jax.ShapeDtypeStruct((M, NOUT), jnp.float32),
        in_specs=[pl.BlockSpec(memory_space=pltpu.VMEM)],
        out_specs=pl.BlockSpec(memory_space=pl.ANY),
        scratch_shapes=[
            pltpu.VMEM((BLK, NOUT), jnp.float32),
            pltpu.VMEM((BLK, NOUT), jnp.float32),
            pltpu.SemaphoreType.DMA((NSEM,)),
            pltpu.SemaphoreType.DMA((NSEM,)),
            pltpu.SemaphoreType.DMA,
        ],
        compiler_params=pltpu.CompilerParams(collective_id=0),
    )(partial)


# device time: 440036 ns/iter; 1.8975x vs baseline; 1.2378x over previous
import functools

import jax
import jax.numpy as jnp
from jax import lax
from jax.experimental import pallas as pl
from jax.experimental.pallas import tpu as pltpu

M = 4096
NOUT = 4096
XS, YS, ZS = 2, 4, 4
NBLK = YS * ZS
BLK = M // NBLK
SLAB = M // YS

W_TOT = 2944
W_A = 1408
W_B = 1536
W_F = NOUT - W_TOT

S_X = 0
S_ZA = 1
S_YB = 4
S_YA = 7
S_ZB = 19
S_F = 31
NSEM = 46


def _fwd_schedule():
    sched = {}
    for i in range(YS):
        for j in range(ZS):
            if i == 0 and j == 0:
                continue
            pa = -1 if i == 0 else (i - 1) * 8 + 2 * j
            pb = -1 if j == 0 else (j - 1) * 8 + 2 * i + 1
            sched.setdefault(max(pa, pb), []).append((i, j))
    return sched


_FWD_AT = _fwd_schedule()


def kernel(dy, W):
    my_y = lax.axis_index("y")
    my_z = lax.axis_index("z")
    r = my_y * ZS + my_z
    dy_rows = lax.dynamic_slice(dy, (r * BLK, 0), (BLK, dy.shape[1]))
    partial = jnp.einsum(
        "mk,nk->mn", dy_rows, W, preferred_element_type=jnp.float32
    )

    def body(p_ref, out_ref, xrecv, sumbuf, send_sems, recv_sems, copy_sem):
        x = lax.axis_index("x")
        y = lax.axis_index("y")
        z = lax.axis_index("z")
        blk = y * ZS + z
        ca = x * W_F
        cb = ca + W_A
        cf = x * W_TOT
        partner = (1 - x, y, z)
        znext = (x, y, (z + 1) % ZS)
        ynext = (x, (y + 1) % YS, z)

        neighbors = [
            partner,
            znext,
            (x, y, (z - 1) % ZS),
            ynext,
            (x, (y - 1) % YS, z),
        ]

        barrier = pltpu.get_barrier_semaphore()
        for nbr in neighbors:
            pl.semaphore_signal(
                barrier, inc=1, device_id=nbr,
                device_id_type=pl.DeviceIdType.MESH,
            )
        pl.semaphore_wait(barrier, len(neighbors))

        def row_of(i, j):
            return ((y - i) % YS) * SLAB + ((z - j) % ZS) * BLK

        def rdma(row0, col0, width, slot, dev):
            ref = out_ref.at[pl.ds(row0, BLK), pl.ds(col0, width)]
            return pltpu.make_async_remote_copy(
                src_ref=ref, dst_ref=ref,
                send_sem=send_sems.at[slot], recv_sem=recv_sems.at[slot],
                device_id=dev, device_id_type=pl.DeviceIdType.MESH,
            )

        xchg = pltpu.make_async_remote_copy(
            src_ref=p_ref, dst_ref=xrecv,
            send_sem=send_sems.at[S_X], recv_sem=recv_sems.at[S_X],
            device_id=partner, device_id_type=pl.DeviceIdType.MESH,
        )
        xchg.start()
        xchg.wait()
        sumbuf[...] = p_ref[...] + xrecv[...]
        cp = pltpu.make_async_copy(
            sumbuf, out_ref.at[pl.ds(blk * BLK, BLK), :], copy_sem
        )
        cp.start()
        cp.wait()

        za = [rdma(row_of(0, h), ca, W_A, S_ZA + h, znext) for h in range(3)]
        yb = [rdma(row_of(h, 0), cb, W_B, S_YB + h, ynext) for h in range(3)]
        ya = [[rdma(row_of(w, k), ca, W_A, S_YA + 4 * w + k, ynext)
               for k in range(4)] for w in range(3)]
        zb = [[rdma(row_of(k, w), cb, W_B, S_ZB + 4 * w + k, znext)
               for k in range(4)] for w in range(3)]
        fwd = {}
        for i in range(YS):
            for j in range(ZS):
                if (i, j) != (0, 0):
                    fwd[(i, j)] = rdma(
                        row_of(i, j), cf, W_F,
                        S_F + i * 4 + j - 1, partner,
                    )

        za[0].start()
        yb[0].start()
        ya[0][0].start()
        zb[0][0].start()

        for h in range(3):
            za[h].wait_recv()
            if h < 2:
                za[h + 1].start()
            ya[0][h + 1].start()
            yb[h].wait_recv()
            if h < 2:
                yb[h + 1].start()
            zb[0][h + 1].start()

        for w in range(3):
            for k in range(4):
                ya[w][k].wait_recv()
                if w < 2:
                    ya[w + 1][k].start()
                for ij in _FWD_AT.get(w * 8 + 2 * k, []):
                    fwd[ij].start()
                zb[w][k].wait_recv()
                if w < 2:
                    zb[w + 1][k].start()
                for ij in _FWD_AT.get(w * 8 + 2 * k + 1, []):
                    fwd[ij].start()

        for f in fwd.values():
            f.wait()
        for d in za + yb:
            d.wait_send()
        for w in range(3):
            for k in range(4):
                ya[w][k].wait_send()
                zb[w][k].wait_send()

        @functools.partial(pl.run_scoped, sem=pltpu.SemaphoreType.REGULAR)
        def _(sem):
            for nbr in neighbors:
                pl.semaphore_signal(
                    sem, inc=1, device_id=nbr,
                    device_id_type=pl.DeviceIdType.MESH,
                )
            pl.semaphore_wait(sem, len(neighbors))

    return pl.pallas_call(
        body,
        out_shape=jax.ShapeDtypeStruct((M, NOUT), jnp.float32),
        in_specs=[pl.BlockSpec(memory_space=pltpu.VMEM)],
        out_specs=pl.BlockSpec(memory_space=pl.ANY),
        scratch_shapes=[
            pltpu.VMEM((BLK, NOUT), jnp.float32),
            pltpu.VMEM((BLK, NOUT), jnp.float32),
            pltpu.SemaphoreType.DMA((NSEM,)),
            pltpu.SemaphoreType.DMA((NSEM,)),
            pltpu.SemaphoreType.DMA,
        ],
        compiler_params=pltpu.CompilerParams(collective_id=0),
    )(partial)


# device time: 425602 ns/iter; 1.9619x vs baseline; 1.0339x over previous
import functools

import jax
import jax.numpy as jnp
from jax import lax
from jax.experimental import pallas as pl
from jax.experimental.pallas import tpu as pltpu

M = 4096
NOUT = 4096
XS, YS, ZS = 2, 4, 4
NBLK = YS * ZS
BLK = M // NBLK
SLAB = M // YS

W_TOT = 2944
W_A = 1408
W_B = 1536
W_F = NOUT - W_TOT

S_X = 0
S_ZA = 1
S_YB = 4
S_YA = 7
S_ZB = 19
S_F = 31
NSEM = 47


def _fwd_schedule():
    sched = {}
    for i in range(YS):
        for j in range(ZS):
            if i == 0 and j == 0:
                continue
            pa = -1 if i == 0 else (i - 1) * 8 + 2 * j
            pb = -1 if j == 0 else (j - 1) * 8 + 2 * i + 1
            sched.setdefault(max(pa, pb), []).append((i, j))
    return sched


_FWD_AT = _fwd_schedule()


def kernel(dy, W):
    my_y = lax.axis_index("y")
    my_z = lax.axis_index("z")
    r = my_y * ZS + my_z
    dy_rows = lax.dynamic_slice(dy, (r * BLK, 0), (BLK, dy.shape[1]))
    partial = jnp.einsum(
        "mk,nk->mn", dy_rows, W, preferred_element_type=jnp.float32
    )

    def body(p_ref, out_ref, xrecv, sumbuf, send_sems, recv_sems, copy_sem):
        x = lax.axis_index("x")
        y = lax.axis_index("y")
        z = lax.axis_index("z")
        blk = y * ZS + z
        ca = x * W_F
        cb = ca + W_A
        cf = x * W_TOT
        partner = (1 - x, y, z)
        znext = (x, y, (z + 1) % ZS)
        ynext = (x, (y + 1) % YS, z)

        neighbors = [
            partner,
            znext,
            (x, y, (z - 1) % ZS),
            ynext,
            (x, (y - 1) % YS, z),
        ]

        barrier = pltpu.get_barrier_semaphore()
        for nbr in neighbors:
            pl.semaphore_signal(
                barrier, inc=1, device_id=nbr,
                device_id_type=pl.DeviceIdType.MESH,
            )
        pl.semaphore_wait(barrier, len(neighbors))

        def row_of(i, j):
            return ((y - i) % YS) * SLAB + ((z - j) % ZS) * BLK

        def rdma(row0, col0, width, slot, dev):
            ref = out_ref.at[pl.ds(row0, BLK), pl.ds(col0, width)]
            return pltpu.make_async_remote_copy(
                src_ref=ref, dst_ref=ref,
                send_sem=send_sems.at[slot], recv_sem=recv_sems.at[slot],
                device_id=dev, device_id_type=pl.DeviceIdType.MESH,
            )

        ca_p = (1 - x) * W_F
        xchg = pltpu.make_async_remote_copy(
            src_ref=p_ref.at[:, pl.ds(ca_p, W_TOT)],
            dst_ref=xrecv.at[:, pl.ds(ca_p, W_TOT)],
            send_sem=send_sems.at[S_X], recv_sem=recv_sems.at[S_X],
            device_id=partner, device_id_type=pl.DeviceIdType.MESH,
        )
        xchg.start()
        xchg.wait()
        sumbuf[...] = p_ref[...] + xrecv[...]
        cp = pltpu.make_async_copy(
            sumbuf.at[:, pl.ds(ca, W_TOT)],
            out_ref.at[pl.ds(blk * BLK, BLK), pl.ds(ca, W_TOT)],
            copy_sem,
        )
        cp.start()

        def rdma_from_sumbuf(row0, col0, width, slot, dev):
            return pltpu.make_async_remote_copy(
                src_ref=sumbuf.at[:, pl.ds(col0, width)],
                dst_ref=out_ref.at[pl.ds(row0, BLK), pl.ds(col0, width)],
                send_sem=send_sems.at[slot], recv_sem=recv_sems.at[slot],
                device_id=dev, device_id_type=pl.DeviceIdType.MESH,
            )

        za = [
            (rdma_from_sumbuf if h == 0 else rdma)(
                row_of(0, h), ca, W_A, S_ZA + h, znext
            )
            for h in range(3)
        ]
        yb = [
            (rdma_from_sumbuf if h == 0 else rdma)(
                row_of(h, 0), cb, W_B, S_YB + h, ynext
            )
            for h in range(3)
        ]
        ya = [[
            (rdma_from_sumbuf if (w, k) == (0, 0) else rdma)(
                row_of(w, k), ca, W_A, S_YA + 4 * w + k, ynext
            )
            for k in range(4)] for w in range(3)]
        zb = [[
            (rdma_from_sumbuf if (w, k) == (0, 0) else rdma)(
                row_of(k, w), cb, W_B, S_ZB + 4 * w + k, znext
            )
            for k in range(4)] for w in range(3)]
        fwd = {}
        for i in range(YS):
            for j in range(ZS):
                fwd[(i, j)] = (rdma_from_sumbuf if (i, j) == (0, 0) else rdma)(
                    row_of(i, j), cf, W_F, S_F + i * 4 + j, partner
                )

        za[0].start()
        yb[0].start()
        ya[0][0].start()
        zb[0][0].start()
        fwd[(0, 0)].start()

        for h in range(3):
            za[h].wait_recv()
            if h < 2:
                za[h + 1].start()
            ya[0][h + 1].start()
            yb[h].wait_recv()
            if h < 2:
                yb[h + 1].start()
            zb[0][h + 1].start()

        for w in range(3):
            for k in range(4):
                ya[w][k].wait_recv()
                if w < 2:
                    ya[w + 1][k].start()
                for ij in _FWD_AT.get(w * 8 + 2 * k, []):
                    fwd[ij].start()
                zb[w][k].wait_recv()
                if w < 2:
                    zb[w + 1][k].start()
                for ij in _FWD_AT.get(w * 8 + 2 * k + 1, []):
                    fwd[ij].start()

        for f in fwd.values():
            f.wait()
        cp.wait()
        for d in za + yb:
            d.wait_send()
        for w in range(3):
            for k in range(4):
                ya[w][k].wait_send()
                zb[w][k].wait_send()

        @functools.partial(pl.run_scoped, sem=pltpu.SemaphoreType.REGULAR)
        def _(sem):
            for nbr in neighbors:
                pl.semaphore_signal(
                    sem, inc=1, device_id=nbr,
                    device_id_type=pl.DeviceIdType.MESH,
                )
            pl.semaphore_wait(sem, len(neighbors))

    return pl.pallas_call(
        body,
        out_shape=jax.ShapeDtypeStruct((M, NOUT), jnp.float32),
        in_specs=[pl.BlockSpec(memory_space=pltpu.VMEM)],
        out_specs=pl.BlockSpec(memory_space=pl.ANY),
        scratch_shapes=[
            pltpu.VMEM((BLK, NOUT), jnp.float32),
            pltpu.VMEM((BLK, NOUT), jnp.float32),
            pltpu.SemaphoreType.DMA((NSEM,)),
            pltpu.SemaphoreType.DMA((NSEM,)),
            pltpu.SemaphoreType.DMA,
        ],
        compiler_params=pltpu.CompilerParams(collective_id=0),
    )(partial)


# device time: 398542 ns/iter; 2.0951x vs baseline; 1.0679x over previous
import functools

import jax
import jax.numpy as jnp
from jax import lax
from jax.experimental import pallas as pl
from jax.experimental.pallas import tpu as pltpu

M = 4096
NOUT = 4096
K = 8192
KT = 2048
XS, YS, ZS = 2, 4, 4
NBLK = YS * ZS
BLK = M // NBLK
SLAB = M // YS

W_TOT = 2944
W_A = 1408
W_B = 1536
W_F = NOUT - W_TOT

S_XA = 0
S_XB = 1
S_ZA = 2
S_YB = 5
S_YA = 8
S_ZB = 20
S_F = 32
NSEM = 48


def _fwd_schedule():
    sched = {}
    for i in range(YS):
        for j in range(ZS):
            if i == 0 and j == 0:
                continue
            pa = -1 if i == 0 else (i - 1) * 8 + 2 * j
            pb = -1 if j == 0 else (j - 1) * 8 + 2 * i + 1
            sched.setdefault(max(pa, pb), []).append((i, j))
    return sched


_FWD_AT = _fwd_schedule()


def kernel(dy, W):

    def body(dy_ref, w_ref, out_ref, dybuf, wbuf, pfull, xrecv, sumbuf,
             send_sems, recv_sems, copy_sem, wsems, dysem):
        x = lax.axis_index("x")
        y = lax.axis_index("y")
        z = lax.axis_index("z")
        blk = y * ZS + z
        ca = x * W_F
        cb = ca + W_A
        cf = x * W_TOT
        ca_p = (1 - x) * W_F
        partner = (1 - x, y, z)
        znext = (x, y, (z + 1) % ZS)
        ynext = (x, (y + 1) % YS, z)

        neighbors = [
            partner,
            znext,
            (x, y, (z - 1) % ZS),
            ynext,
            (x, (y - 1) % YS, z),
        ]

        barrier = pltpu.get_barrier_semaphore()
        for nbr in neighbors:
            pl.semaphore_signal(
                barrier, inc=1, device_id=nbr,
                device_id_type=pl.DeviceIdType.MESH,
            )
        pl.semaphore_wait(barrier, len(neighbors))

        pieces = [(ca_p, W_A), (ca_p + W_A, W_B), (cf, W_F)]
        tiles = [(p, k) for p in range(3) for k in range(K // KT)]

        dycp = pltpu.make_async_copy(
            dy_ref.at[pl.ds(blk * BLK, BLK), :], dybuf, dysem
        )
        dycp.start()

        def wtile_copy(t, buf):
            ps, pw = pieces[tiles[t][0]]
            k = tiles[t][1]
            return pltpu.make_async_copy(
                w_ref.at[pl.ds(ps, pw), pl.ds(k * KT, KT)],
                wbuf.at[buf, pl.ds(0, pw), :],
                wsems.at[buf],
            )

        wcp = wtile_copy(0, 0)
        wcp.start()
        dycp.wait()

        xchg = []
        for t, (p, k) in enumerate(tiles):
            wtile_copy(t, t % 2).wait()
            if t + 1 < len(tiles):
                wtile_copy(t + 1, (t + 1) % 2).start()
            ps, pw = pieces[p]
            d = lax.dot_general(
                dybuf[:, pl.ds(k * KT, KT)],
                wbuf[t % 2, pl.ds(0, pw), :],
                (((1,), (1,)), ((), ())),
                preferred_element_type=jnp.float32,
            )
            if k == 0:
                pfull[:, pl.ds(ps, pw)] = d
            else:
                pfull[:, pl.ds(ps, pw)] = pfull[:, pl.ds(ps, pw)] + d
            if k == K // KT - 1 and p < 2:
                ex = pltpu.make_async_remote_copy(
                    src_ref=pfull.at[:, pl.ds(ps, pw)],
                    dst_ref=xrecv.at[:, pl.ds(ps, pw)],
                    send_sem=send_sems.at[S_XA + p],
                    recv_sem=recv_sems.at[S_XA + p],
                    device_id=partner,
                    device_id_type=pl.DeviceIdType.MESH,
                )
                ex.start()
                xchg.append(ex)

        def row_of(i, j):
            return ((y - i) % YS) * SLAB + ((z - j) % ZS) * BLK

        def rdma(row0, col0, width, slot, dev):
            ref = out_ref.at[pl.ds(row0, BLK), pl.ds(col0, width)]
            return pltpu.make_async_remote_copy(
                src_ref=ref, dst_ref=ref,
                send_sem=send_sems.at[slot], recv_sem=recv_sems.at[slot],
                device_id=dev, device_id_type=pl.DeviceIdType.MESH,
            )

        def rdma_from_sumbuf(row0, col0, width, slot, dev):
            return pltpu.make_async_remote_copy(
                src_ref=sumbuf.at[:, pl.ds(col0, width)],
                dst_ref=out_ref.at[pl.ds(row0, BLK), pl.ds(col0, width)],
                send_sem=send_sems.at[slot], recv_sem=recv_sems.at[slot],
                device_id=dev, device_id_type=pl.DeviceIdType.MESH,
            )

        za = [
            (rdma_from_sumbuf if h == 0 else rdma)(
                row_of(0, h), ca, W_A, S_ZA + h, znext
            )
            for h in range(3)
        ]
        yb = [
            (rdma_from_sumbuf if h == 0 else rdma)(
                row_of(h, 0), cb, W_B, S_YB + h, ynext
            )
            for h in range(3)
        ]
        ya = [[
            (rdma_from_sumbuf if (w, k) == (0, 0) else rdma)(
                row_of(w, k), ca, W_A, S_YA + 4 * w + k, ynext
            )
            for k in range(4)] for w in range(3)]
        zb = [[
            (rdma_from_sumbuf if (w, k) == (0, 0) else rdma)(
                row_of(k, w), cb, W_B, S_ZB + 4 * w + k, znext
            )
            for k in range(4)] for w in range(3)]
        fwd = {}
        for i in range(YS):
            for j in range(ZS):
                fwd[(i, j)] = (rdma_from_sumbuf if (i, j) == (0, 0) else rdma)(
                    row_of(i, j), cf, W_F, S_F + i * 4 + j, partner
                )

        xchg[0].wait()
        sumbuf[:, pl.ds(ca, W_A)] = (
            pfull[:, pl.ds(ca, W_A)] + xrecv[:, pl.ds(ca, W_A)]
        )
        za[0].start()
        ya[0][0].start()
        cpa = pltpu.make_async_copy(
            sumbuf.at[:, pl.ds(ca, W_A)],
            out_ref.at[pl.ds(blk * BLK, BLK), pl.ds(ca, W_A)],
            copy_sem,
        )
        cpa.start()

        xchg[1].wait()
        sumbuf[:, pl.ds(cb, W_B)] = (
            pfull[:, pl.ds(cb, W_B)] + xrecv[:, pl.ds(cb, W_B)]
        )
        yb[0].start()
        zb[0][0].start()
        fwd[(0, 0)].start()
        cpa.wait()
        cpb = pltpu.make_async_copy(
            sumbuf.at[:, pl.ds(cb, W_B)],
            out_ref.at[pl.ds(blk * BLK, BLK), pl.ds(cb, W_B)],
            copy_sem,
        )
        cpb.start()

        for h in range(3):
            za[h].wait_recv()
            if h < 2:
                za[h + 1].start()
            ya[0][h + 1].start()
            yb[h].wait_recv()
            if h < 2:
                yb[h + 1].start()
            zb[0][h + 1].start()

        for w in range(3):
            for k in range(4):
                ya[w][k].wait_recv()
                if w < 2:
                    ya[w + 1][k].start()
                for ij in _FWD_AT.get(w * 8 + 2 * k, []):
                    fwd[ij].start()
                zb[w][k].wait_recv()
                if w < 2:
                    zb[w + 1][k].start()
                for ij in _FWD_AT.get(w * 8 + 2 * k + 1, []):
                    fwd[ij].start()

        for f in fwd.values():
            f.wait()
        cpb.wait()
        for d in za + yb:
            d.wait_send()
        for w in range(3):
            for k in range(4):
                ya[w][k].wait_send()
                zb[w][k].wait_send()

        @functools.partial(pl.run_scoped, sem=pltpu.SemaphoreType.REGULAR)
        def _(sem):
            for nbr in neighbors:
                pl.semaphore_signal(
                    sem, inc=1, device_id=nbr,
                    device_id_type=pl.DeviceIdType.MESH,
                )
            pl.semaphore_wait(sem, len(neighbors))

    return pl.pallas_call(
        body,
        out_shape=jax.ShapeDtypeStruct((M, NOUT), jnp.float32),
        in_specs=[
            pl.BlockSpec(memory_space=pl.ANY),
            pl.BlockSpec(memory_space=pl.ANY),
        ],
        out_specs=pl.BlockSpec(memory_space=pl.ANY),
        scratch_shapes=[
            pltpu.VMEM((BLK, K), jnp.float32),
            pltpu.VMEM((2, W_B, KT), jnp.float32),
            pltpu.VMEM((BLK, NOUT), jnp.float32),
            pltpu.VMEM((BLK, NOUT), jnp.float32),
            pltpu.VMEM((BLK, NOUT), jnp.float32),
            pltpu.SemaphoreType.DMA((NSEM,)),
            pltpu.SemaphoreType.DMA((NSEM,)),
            pltpu.SemaphoreType.DMA,
            pltpu.SemaphoreType.DMA((2,)),
            pltpu.SemaphoreType.DMA,
        ],
        compiler_params=pltpu.CompilerParams(
            collective_id=0,
            vmem_limit_bytes=60 * 1024 * 1024,
        ),
    )(dy, W)
